# baseline (speedup 1.0000x reference)
import functools

import jax
import jax.numpy as jnp
from jax import lax
from jax.experimental import pallas as pl
from jax.experimental.pallas import tpu as pltpu
from jax.experimental.pallas import tpu_sc as plsc

NC = 2
NS = 16
NW = NC * NS
KCH = 128
NBUF = 12
PREF = 6


def _pad_rows(n):
  per_tile = -(-(n + 16) // (NS * 16)) * 16
  return per_tile * NS, per_tile


def _chunks(total, step):
  return [(o, min(step, total - o)) for o in range(0, total, step)]


def _load_repack(e_hbm, row, epw, wid, flatv, idx2d, trash_base):
  ngrp = -(-epw // 16)
  lane = lax.broadcasted_iota(jnp.int32, (16,), 0)
  trash = trash_base + lane
  if epw % 16:
    flatv[pl.ds(ngrp * 16 - 16, 16)] = trash
  for g in range(ngrp, (idx2d.shape[0] * KCH) // 16):
    idx2d[g // (KCH // 16), pl.ds((g % (KCH // 16)) * 16, 16)] = trash
  pltpu.sync_copy(e_hbm.at[row, pl.ds(wid * epw, epw)],
                  flatv.at[pl.ds(0, epw)])
  for g in range(ngrp):
    idx2d[g // (KCH // 16), pl.ds((g % (KCH // 16)) * 16, 16)] = (
        flatv[pl.ds(g * 16, 16)])


def _make_deg_kernel(n, e):
  nr, rpt = _pad_rows(n)
  epw = e // NW
  ch = -(-epw // KCH)
  mesh = plsc.VectorSubcoreMesh(core_axis_name="c", subcore_axis_name="s")

  @functools.partial(
      pl.kernel,
      out_type=jax.ShapeDtypeStruct((NC * nr,), jnp.float32),
      mesh=mesh,
      scratch_types=[
          pltpu.VMEM((-(-epw // 16) * 16,), jnp.int32),
          pltpu.VMEM((ch, KCH), jnp.int32),
          pltpu.VMEM((KCH,), jnp.float32),
          pltpu.VMEM((rpt,), jnp.float32),
          pltpu.VMEM_SHARED((nr,), jnp.float32),
          pltpu.SemaphoreType.DMA,
      ],
      compiler_params=pltpu.CompilerParams(use_tc_tiling_on_sc=False),
  )
  def deg_kernel(e_hbm, out_hbm, flatv, dstv, onesv, stagev, accum, sem):
    c = lax.axis_index("c")
    s = lax.axis_index("s")
    wid = c * NS + s
    one = jnp.ones((16,), jnp.float32)
    zero = jnp.zeros((16,), jnp.float32)
    for r in range(KCH // 16):
      onesv[pl.ds(r * 16, 16)] = one
    for r in range(rpt // 16):
      stagev[pl.ds(r * 16, 16)] = zero
    pltpu.sync_copy(stagev, accum.at[pl.ds(s * rpt, rpt)])
    _load_repack(e_hbm, 1, epw, wid, flatv, dstv, n)
    plsc.subcore_barrier()
    descs = [
        pltpu.async_copy(onesv, accum.at[dstv.at[j]], sem, add=True)
        for j in range(ch)
    ]
    for d in descs:
      d.wait()
    plsc.subcore_barrier()
    pltpu.sync_copy(accum.at[pl.ds(s * rpt, rpt)], stagev)
    pltpu.sync_copy(stagev, out_hbm.at[pl.ds(c * nr + s * rpt, rpt)])

  return deg_kernel


def _make_msg_kernel(n, w, e):
  nr, rpt = _pad_rows(n)
  epw = e // NW
  ch = -(-epw // KCH)
  mesh = plsc.VectorSubcoreMesh(core_axis_name="c", subcore_axis_name="s")

  @functools.partial(
      pl.kernel,
      out_type=jax.ShapeDtypeStruct((NC, nr, w), jnp.float32),
      mesh=mesh,
      scratch_types=(
          [pltpu.VMEM((-(-epw // 16) * 16,), jnp.int32),
           pltpu.VMEM((ch, KCH), jnp.int32),
           pltpu.VMEM((ch, KCH), jnp.int32),
           pltpu.VMEM((NBUF, KCH, w), jnp.float32),
           pltpu.VMEM((rpt, w), jnp.float32),
           pltpu.VMEM_SHARED((nr, w), jnp.float32)]
          + [pltpu.SemaphoreType.DMA] * (2 * NBUF)
      ),
      compiler_params=pltpu.CompilerParams(use_tc_tiling_on_sc=False),
  )
  def msg_kernel(h_hbm, e_hbm, out_hbm,
                 flatv, srcv, dstv, rowsv, stagev, accum, *sems):
    gsem = sems[:NBUF]
    ssem = sems[NBUF:]
    c = lax.axis_index("c")
    s = lax.axis_index("s")
    wid = c * NS + s
    zero = jnp.zeros((16,), jnp.float32)
    for r in range(KCH):
      for cc in range(w // 16):
        stagev[r, pl.ds(cc * 16, 16)] = zero
    for off, sz in _chunks(rpt, KCH):
      pltpu.sync_copy(stagev.at[pl.ds(0, sz)],
                      accum.at[pl.ds(s * rpt + off, sz)])
    _load_repack(e_hbm, 0, epw, wid, flatv, srcv, wid * 16)
    _load_repack(e_hbm, 1, epw, wid, flatv, dstv, n)
    plsc.subcore_barrier()

    gd = [None] * NBUF
    sd = [None] * NBUF
    for j in range(min(PREF, ch)):
      gd[j % NBUF] = pltpu.async_copy(
          h_hbm.at[srcv.at[j]], rowsv.at[j % NBUF], gsem[j % NBUF])
    for j in range(ch):
      jn = j + PREF
      if jn < ch:
        bn = jn % NBUF
        if sd[bn] is not None:
          sd[bn].wait()
          sd[bn] = None
        gd[bn] = pltpu.async_copy(
            h_hbm.at[srcv.at[jn]], rowsv.at[bn], gsem[bn])
      b = j % NBUF
      gd[b].wait()
      sd[b] = pltpu.async_copy(
          rowsv.at[b], accum.at[dstv.at[j]], ssem[b], add=True)
    for b in range(NBUF):
      if sd[b] is not None:
        sd[b].wait()
    plsc.subcore_barrier()
    pltpu.sync_copy(accum.at[pl.ds(s * rpt, rpt)], stagev)
    pltpu.sync_copy(stagev, out_hbm.at[c, pl.ds(s * rpt, rpt)])

  return msg_kernel


def _mm1_body(x_ref, w_ref, degp_ref, h1s_ref, dinv_ref):
  deg = 1.0 + degp_ref[0, :] + degp_ref[1, :]
  dinv = lax.rsqrt(deg)[:, None]
  dinv_ref[...] = dinv
  h1 = jnp.dot(x_ref[...], w_ref[...], preferred_element_type=jnp.float32)
  h1s_ref[...] = h1 * dinv


def _mid_body(p_ref, h1_ref, dinv_ref, b1_ref, w2_ref, h2s_ref):
  out1 = p_ref[0, :, :] + p_ref[1, :, :] + h1_ref[...]
  h2 = jnp.maximum(out1 * dinv_ref[...] + b1_ref[...], 0.0)
  h2s_ref[...] = jnp.dot(h2, w2_ref[...],
                         preferred_element_type=jnp.float32) * dinv_ref[...]


def _heads_body(mri_ref, cog_ref, clin_ref, gen_ref,
                wm_ref, bm_ref, wc_ref, bc_ref, wcl_ref, bcl_ref,
                wg_ref, bg_ref, mm_ref):
  mf = jnp.maximum(jnp.dot(mri_ref[...], wm_ref[...],
                           preferred_element_type=jnp.float32) + bm_ref[...], 0.0)
  cf = jnp.maximum(jnp.dot(cog_ref[...], wc_ref[...],
                           preferred_element_type=jnp.float32) + bc_ref[...], 0.0)
  clf = jnp.maximum(jnp.dot(clin_ref[...], wcl_ref[...],
                            preferred_element_type=jnp.float32) + bcl_ref[...], 0.0)
  gnf = jnp.maximum(jnp.dot(gen_ref[...], wg_ref[...],
                            preferred_element_type=jnp.float32) + bg_ref[...], 0.0)
  mm_ref[...] = jnp.concatenate([mf, cf, clf, gnf], axis=1)


def _final_body(q_ref, h2_ref, dinv_ref, b2_ref, batch_ref, mm_ref,
                wf1_ref, bf1_ref, wf2_ref, bf2_ref,
                o_ref, *, n, nb):
  out2 = q_ref[0, :n, :] + q_ref[1, :n, :] + h2_ref[...]
  gf = jnp.maximum(out2 * dinv_ref[...] + b2_ref[...], 0.0)
  seg = lax.broadcasted_iota(jnp.int32, (nb, n), 0)
  maskf = (seg == batch_ref[...]).astype(jnp.float32)
  sums = jnp.dot(maskf, gf, preferred_element_type=jnp.float32)
  cnt = jnp.sum(maskf, axis=1, keepdims=True)
  graph_emb = sums / jnp.maximum(cnt, 1.0)
  comb = jnp.concatenate([graph_emb, mm_ref[...]], axis=1)
  hid = jnp.maximum(jnp.dot(comb, wf1_ref[...],
                            preferred_element_type=jnp.float32) + bf1_ref[...], 0.0)
  logits = jnp.dot(hid, wf2_ref[...],
                   preferred_element_type=jnp.float32) + bf2_ref[...]
  m = jnp.max(logits, axis=1, keepdims=True)
  z = logits - m
  lse = jnp.log(jnp.sum(jnp.exp(z), axis=1, keepdims=True))
  o_ref[...] = z - lse


def kernel(x, edge_index, batch, mri_features, cog_features, clin_features,
           genetic_features, W1, b1, W2, b2, Wm, bm, Wc, bc, Wcl, bcl,
           Wg, bg, Wf1, bf1, Wf2, bf2):
  n, _ = x.shape
  e = edge_index.shape[1]
  nb = mri_features.shape[0]
  nr, _ = _pad_rows(n)

  if e % NW != 0 or (e // NW) % 8 != 0:
    epw = -(-e // NW)
    epw += (-epw) % 8
    pad = NW * epw - e
    pad_col = jnp.stack([jnp.arange(pad, dtype=jnp.int32) % n,
                         n + (jnp.arange(pad, dtype=jnp.int32) % 16)])
    edge_index = jnp.concatenate([edge_index, pad_col], axis=1)
    e = NW * epw

  d = x.shape[1]
  mm = pl.pallas_call(
      _heads_body,
      out_shape=jax.ShapeDtypeStruct((nb, 32), jnp.float32),
  )(mri_features, cog_features, clin_features, genetic_features,
    Wm, bm.reshape(1, -1), Wc, bc.reshape(1, -1), Wcl, bcl.reshape(1, -1),
    Wg, bg.reshape(1, -1))

  degp = _make_deg_kernel(n, e)(edge_index).reshape(2, nr)

  bm_rows = 1024
  grid = -(-n // bm_rows)
  H1, dinv = pl.pallas_call(
      _mm1_body,
      grid=(grid,),
      in_specs=[
          pl.BlockSpec((bm_rows, d), lambda i: (i, 0)),
          pl.BlockSpec((d, 32), lambda i: (0, 0)),
          pl.BlockSpec((2, bm_rows), lambda i: (0, i)),
      ],
      out_specs=(pl.BlockSpec((bm_rows, 32), lambda i: (i, 0)),
                 pl.BlockSpec((bm_rows, 1), lambda i: (i, 0))),
      out_shape=(jax.ShapeDtypeStruct((n, 32), jnp.float32),
                 jax.ShapeDtypeStruct((n, 1), jnp.float32)),
  )(x, W1, degp)

  P = _make_msg_kernel(n, 32, e)(H1, edge_index)
  H2 = pl.pallas_call(
      _mid_body,
      grid=(grid,),
      in_specs=[
          pl.BlockSpec((2, bm_rows, 32), lambda i: (0, i, 0)),
          pl.BlockSpec((bm_rows, 32), lambda i: (i, 0)),
          pl.BlockSpec((bm_rows, 1), lambda i: (i, 0)),
          pl.BlockSpec((1, 32), lambda i: (0, 0)),
          pl.BlockSpec((32, 16), lambda i: (0, 0)),
      ],
      out_specs=pl.BlockSpec((bm_rows, 16), lambda i: (i, 0)),
      out_shape=jax.ShapeDtypeStruct((n, 16), jnp.float32),
  )(P, H1, dinv, b1.reshape(1, 32), W2)

  Q = _make_msg_kernel(n, 16, e)(H2, edge_index)
  out = pl.pallas_call(
      functools.partial(_final_body, n=n, nb=nb),
      out_shape=jax.ShapeDtypeStruct((nb, 3), jnp.float32),
  )(Q, H2, dinv, b2.reshape(1, 16), batch.reshape(1, n), mm,
    Wf1, bf1.reshape(1, -1), Wf2, bf2.reshape(1, -1))
  return out


# speedup vs baseline: 1.0384x; 1.0384x over previous
import functools

import jax
import jax.numpy as jnp
from jax import lax
from jax.experimental import pallas as pl
from jax.experimental.pallas import tpu as pltpu
from jax.experimental.pallas import tpu_sc as plsc

NC = 2
NS = 16
NW = NC * NS
KCH = 128
NBUF = 12
PREF = 6


def _pad_rows(n):
  per_tile = -(-(n + 16) // (NS * 16)) * 16
  return per_tile * NS, per_tile


def _chunks(total, step):
  return [(o, min(step, total - o)) for o in range(0, total, step)]


def _load_repack(e_hbm, row, epw, wid, flatv, idx2d, trash_base):
  ngrp = -(-epw // 16)
  lane = lax.broadcasted_iota(jnp.int32, (16,), 0)
  trash = trash_base + lane
  if epw % 16:
    flatv[pl.ds(ngrp * 16 - 16, 16)] = trash
  for g in range(ngrp, (idx2d.shape[0] * KCH) // 16):
    idx2d[g // (KCH // 16), pl.ds((g % (KCH // 16)) * 16, 16)] = trash
  pltpu.sync_copy(e_hbm.at[row, pl.ds(wid * epw, epw)],
                  flatv.at[pl.ds(0, epw)])
  for g in range(ngrp):
    idx2d[g // (KCH // 16), pl.ds((g % (KCH // 16)) * 16, 16)] = (
        flatv[pl.ds(g * 16, 16)])


def _make_deg_kernel(n, e):
  nr, rpt = _pad_rows(n)
  epw = e // NW
  ch = -(-epw // KCH)
  mesh = plsc.VectorSubcoreMesh(core_axis_name="c", subcore_axis_name="s")

  @functools.partial(
      pl.kernel,
      out_type=jax.ShapeDtypeStruct((NC * nr,), jnp.float32),
      mesh=mesh,
      scratch_types=[
          pltpu.VMEM((-(-epw // 16) * 16,), jnp.int32),
          pltpu.VMEM((ch, KCH), jnp.int32),
          pltpu.VMEM((KCH,), jnp.float32),
          pltpu.VMEM((rpt,), jnp.float32),
          pltpu.VMEM_SHARED((nr,), jnp.float32),
          pltpu.SemaphoreType.DMA,
      ],
      compiler_params=pltpu.CompilerParams(use_tc_tiling_on_sc=False),
  )
  def deg_kernel(e_hbm, out_hbm, flatv, dstv, onesv, stagev, accum, sem):
    c = lax.axis_index("c")
    s = lax.axis_index("s")
    wid = c * NS + s
    one = jnp.ones((16,), jnp.float32)
    zero = jnp.zeros((16,), jnp.float32)
    for r in range(KCH // 16):
      onesv[pl.ds(r * 16, 16)] = one
    for r in range(rpt // 16):
      stagev[pl.ds(r * 16, 16)] = zero
    pltpu.sync_copy(stagev, accum.at[pl.ds(s * rpt, rpt)])
    _load_repack(e_hbm, 1, epw, wid, flatv, dstv, n)
    plsc.subcore_barrier()
    descs = [
        pltpu.async_copy(onesv, accum.at[dstv.at[j]], sem, add=True)
        for j in range(ch)
    ]
    for d in descs:
      d.wait()
    plsc.subcore_barrier()
    pltpu.sync_copy(accum.at[pl.ds(s * rpt, rpt)], stagev)
    pltpu.sync_copy(stagev, out_hbm.at[pl.ds(c * nr + s * rpt, rpt)])

  return deg_kernel


def _make_msg_kernel(n, w, e):
  nr, rpt = _pad_rows(n)
  epw = e // NW
  ch = -(-epw // KCH)
  mesh = plsc.VectorSubcoreMesh(core_axis_name="c", subcore_axis_name="s")

  @functools.partial(
      pl.kernel,
      out_type=jax.ShapeDtypeStruct((NC, nr, w), jnp.float32),
      mesh=mesh,
      scratch_types=(
          [pltpu.VMEM((-(-epw // 16) * 16,), jnp.int32),
           pltpu.VMEM((ch, KCH), jnp.int32),
           pltpu.VMEM((ch, KCH), jnp.int32),
           pltpu.VMEM((NBUF, KCH, w), jnp.float32),
           pltpu.VMEM((rpt, w), jnp.float32),
           pltpu.VMEM_SHARED((nr, w), jnp.float32)]
          + [pltpu.SemaphoreType.DMA] * (2 * NBUF)
      ),
      compiler_params=pltpu.CompilerParams(use_tc_tiling_on_sc=False),
  )
  def msg_kernel(h_hbm, e_hbm, out_hbm,
                 flatv, srcv, dstv, rowsv, stagev, accum, *sems):
    gsem = sems[:NBUF]
    ssem = sems[NBUF:]
    c = lax.axis_index("c")
    s = lax.axis_index("s")
    wid = c * NS + s
    zero = jnp.zeros((16,), jnp.float32)
    for r in range(KCH):
      for cc in range(w // 16):
        stagev[r, pl.ds(cc * 16, 16)] = zero
    for off, sz in _chunks(rpt, KCH):
      pltpu.sync_copy(stagev.at[pl.ds(0, sz)],
                      accum.at[pl.ds(s * rpt + off, sz)])
    _load_repack(e_hbm, 0, epw, wid, flatv, srcv, wid * 16)
    _load_repack(e_hbm, 1, epw, wid, flatv, dstv, n)
    plsc.subcore_barrier()

    gd = [None] * NBUF
    sd = [None] * NBUF
    for j in range(min(PREF, ch)):
      gd[j % NBUF] = pltpu.async_copy(
          h_hbm.at[srcv.at[j]], rowsv.at[j % NBUF], gsem[j % NBUF])
    for j in range(ch):
      jn = j + PREF
      if jn < ch:
        bn = jn % NBUF
        if sd[bn] is not None:
          sd[bn].wait()
          sd[bn] = None
        gd[bn] = pltpu.async_copy(
            h_hbm.at[srcv.at[jn]], rowsv.at[bn], gsem[bn])
      b = j % NBUF
      gd[b].wait()
      sd[b] = pltpu.async_copy(
          rowsv.at[b], accum.at[dstv.at[j]], ssem[b], add=True)
    for b in range(NBUF):
      if sd[b] is not None:
        sd[b].wait()
    plsc.subcore_barrier()
    pltpu.sync_copy(accum.at[pl.ds(s * rpt, rpt)], stagev)
    pltpu.sync_copy(stagev, out_hbm.at[c, pl.ds(s * rpt, rpt)])

  return msg_kernel


def _mm1_body(x_ref, w_ref, degp_ref, h1s_ref, dinv_ref, *, n):
  deg = 1.0 + degp_ref[0, :n] + degp_ref[1, :n]
  dinv = lax.rsqrt(deg)[:, None]
  dinv_ref[...] = dinv
  h1 = jnp.dot(x_ref[...], w_ref[...], preferred_element_type=jnp.float32)
  h1s_ref[...] = h1 * dinv


def _mid_body(p_ref, h1_ref, dinv_ref, b1_ref, w2_ref, h2s_ref, *, n):
  out1 = p_ref[0, :n, :] + p_ref[1, :n, :] + h1_ref[...]
  h2 = jnp.maximum(out1 * dinv_ref[...] + b1_ref[...], 0.0)
  h2s_ref[...] = jnp.dot(h2, w2_ref[...],
                         preferred_element_type=jnp.float32) * dinv_ref[...]


def _heads_body(mri_ref, cog_ref, clin_ref, gen_ref,
                wm_ref, bm_ref, wc_ref, bc_ref, wcl_ref, bcl_ref,
                wg_ref, bg_ref, mm_ref):
  mf = jnp.maximum(jnp.dot(mri_ref[...], wm_ref[...],
                           preferred_element_type=jnp.float32) + bm_ref[...], 0.0)
  cf = jnp.maximum(jnp.dot(cog_ref[...], wc_ref[...],
                           preferred_element_type=jnp.float32) + bc_ref[...], 0.0)
  clf = jnp.maximum(jnp.dot(clin_ref[...], wcl_ref[...],
                            preferred_element_type=jnp.float32) + bcl_ref[...], 0.0)
  gnf = jnp.maximum(jnp.dot(gen_ref[...], wg_ref[...],
                            preferred_element_type=jnp.float32) + bg_ref[...], 0.0)
  mm_ref[...] = jnp.concatenate([mf, cf, clf, gnf], axis=1)


def _final_body(q_ref, h2_ref, dinv_ref, b2_ref, batch_ref, mm_ref,
                wf1_ref, bf1_ref, wf2_ref, bf2_ref,
                o_ref, *, n, nb):
  out2 = q_ref[0, :n, :] + q_ref[1, :n, :] + h2_ref[...]
  gf = jnp.maximum(out2 * dinv_ref[...] + b2_ref[...], 0.0)
  seg = lax.broadcasted_iota(jnp.int32, (nb, n), 0)
  maskf = (seg == batch_ref[...]).astype(jnp.float32)
  sums = jnp.dot(maskf, gf, preferred_element_type=jnp.float32)
  cnt = jnp.sum(maskf, axis=1, keepdims=True)
  graph_emb = sums / jnp.maximum(cnt, 1.0)
  comb = jnp.concatenate([graph_emb, mm_ref[...]], axis=1)
  hid = jnp.maximum(jnp.dot(comb, wf1_ref[...],
                            preferred_element_type=jnp.float32) + bf1_ref[...], 0.0)
  logits = jnp.dot(hid, wf2_ref[...],
                   preferred_element_type=jnp.float32) + bf2_ref[...]
  m = jnp.max(logits, axis=1, keepdims=True)
  z = logits - m
  lse = jnp.log(jnp.sum(jnp.exp(z), axis=1, keepdims=True))
  o_ref[...] = z - lse


def kernel(x, edge_index, batch, mri_features, cog_features, clin_features,
           genetic_features, W1, b1, W2, b2, Wm, bm, Wc, bc, Wcl, bcl,
           Wg, bg, Wf1, bf1, Wf2, bf2):
  n, _ = x.shape
  e = edge_index.shape[1]
  nb = mri_features.shape[0]
  nr, _ = _pad_rows(n)

  if e % NW != 0 or (e // NW) % 8 != 0:
    epw = -(-e // NW)
    epw += (-epw) % 8
    pad = NW * epw - e
    pad_col = jnp.stack([jnp.arange(pad, dtype=jnp.int32) % n,
                         n + (jnp.arange(pad, dtype=jnp.int32) % 16)])
    edge_index = jnp.concatenate([edge_index, pad_col], axis=1)
    e = NW * epw

  d = x.shape[1]
  mm = pl.pallas_call(
      _heads_body,
      out_shape=jax.ShapeDtypeStruct((nb, 32), jnp.float32),
  )(mri_features, cog_features, clin_features, genetic_features,
    Wm, bm.reshape(1, -1), Wc, bc.reshape(1, -1), Wcl, bcl.reshape(1, -1),
    Wg, bg.reshape(1, -1))

  degp = _make_deg_kernel(n, e)(edge_index).reshape(2, nr)

  H1, dinv = pl.pallas_call(
      functools.partial(_mm1_body, n=n),
      out_shape=(jax.ShapeDtypeStruct((n, 32), jnp.float32),
                 jax.ShapeDtypeStruct((n, 1), jnp.float32)),
  )(x, W1, degp)

  P = _make_msg_kernel(n, 32, e)(H1, edge_index)
  H2 = pl.pallas_call(
      functools.partial(_mid_body, n=n),
      out_shape=jax.ShapeDtypeStruct((n, 16), jnp.float32),
  )(P, H1, dinv, b1.reshape(1, 32), W2)

  Q = _make_msg_kernel(n, 16, e)(H2, edge_index)
  out = pl.pallas_call(
      functools.partial(_final_body, n=n, nb=nb),
      out_shape=jax.ShapeDtypeStruct((nb, 3), jnp.float32),
  )(Q, H2, dinv, b2.reshape(1, 16), batch.reshape(1, n), mm,
    Wf1, bf1.reshape(1, -1), Wf2, bf2.reshape(1, -1))
  return out


# speedup vs baseline: 1.1234x; 1.0818x over previous
import functools

import jax
import jax.numpy as jnp
from jax import lax
from jax.experimental import pallas as pl
from jax.experimental.pallas import tpu as pltpu
from jax.experimental.pallas import tpu_sc as plsc

NC = 2
NS = 16
NW = NC * NS
KCH = 128
NBUF = 12
PREF = 6


def _pad_rows(n):
  per_tile = -(-(n + 16) // (NS * 16)) * 16
  return per_tile * NS, per_tile


def _chunks(total, step):
  return [(o, min(step, total - o)) for o in range(0, total, step)]


def _load_repack(e_hbm, row, epw, wid, flatv, idx2d, trash_base):
  ngrp = -(-epw // 16)
  lane = lax.broadcasted_iota(jnp.int32, (16,), 0)
  trash = trash_base + lane
  if epw % 16:
    flatv[pl.ds(ngrp * 16 - 16, 16)] = trash
  for g in range(ngrp, (idx2d.shape[0] * KCH) // 16):
    idx2d[g // (KCH // 16), pl.ds((g % (KCH // 16)) * 16, 16)] = trash
  pltpu.sync_copy(e_hbm.at[row, pl.ds(wid * epw, epw)],
                  flatv.at[pl.ds(0, epw)])
  for g in range(ngrp):
    idx2d[g // (KCH // 16), pl.ds((g % (KCH // 16)) * 16, 16)] = (
        flatv[pl.ds(g * 16, 16)])


def _make_deg_kernel(n, e):
  nr, rpt = _pad_rows(n)
  epw = e // NW
  ch = -(-epw // KCH)
  mesh = plsc.VectorSubcoreMesh(core_axis_name="c", subcore_axis_name="s")

  @functools.partial(
      pl.kernel,
      out_type=jax.ShapeDtypeStruct((NC * nr,), jnp.float32),
      mesh=mesh,
      scratch_types=[
          pltpu.VMEM((-(-epw // 16) * 16,), jnp.int32),
          pltpu.VMEM((ch, KCH), jnp.int32),
          pltpu.VMEM((KCH,), jnp.float32),
          pltpu.VMEM((rpt,), jnp.float32),
          pltpu.VMEM_SHARED((nr,), jnp.float32),
          pltpu.SemaphoreType.DMA,
      ],
      compiler_params=pltpu.CompilerParams(use_tc_tiling_on_sc=False),
  )
  def deg_kernel(e_hbm, out_hbm, flatv, dstv, onesv, stagev, accum, sem):
    c = lax.axis_index("c")
    s = lax.axis_index("s")
    wid = c * NS + s
    one = jnp.ones((16,), jnp.float32)
    zero = jnp.zeros((16,), jnp.float32)
    for r in range(KCH // 16):
      onesv[pl.ds(r * 16, 16)] = one
    for r in range(rpt // 16):
      stagev[pl.ds(r * 16, 16)] = zero
    pltpu.sync_copy(stagev, accum.at[pl.ds(s * rpt, rpt)])
    _load_repack(e_hbm, 1, epw, wid, flatv, dstv, n)
    plsc.subcore_barrier()
    descs = [
        pltpu.async_copy(onesv, accum.at[dstv.at[j]], sem, add=True)
        for j in range(ch)
    ]
    for d in descs:
      d.wait()
    plsc.subcore_barrier()
    pltpu.sync_copy(accum.at[pl.ds(s * rpt, rpt)], stagev)
    pltpu.sync_copy(stagev, out_hbm.at[pl.ds(c * nr + s * rpt, rpt)])

  return deg_kernel


def _make_msg_kernel(n, w, e, dtype=jnp.float32):
  nr, rpt = _pad_rows(n)
  epw = e // NW
  ch = -(-epw // KCH)
  lanes = 32 if dtype == jnp.bfloat16 else 16
  mesh = plsc.VectorSubcoreMesh(core_axis_name="c", subcore_axis_name="s")

  @functools.partial(
      pl.kernel,
      out_type=jax.ShapeDtypeStruct((NC, nr, w), dtype),
      mesh=mesh,
      scratch_types=(
          [pltpu.VMEM((-(-epw // 16) * 16,), jnp.int32),
           pltpu.VMEM((ch, KCH), jnp.int32),
           pltpu.VMEM((ch, KCH), jnp.int32),
           pltpu.VMEM((NBUF, KCH, w), dtype),
           pltpu.VMEM((rpt, w), dtype),
           pltpu.VMEM_SHARED((nr, w), dtype)]
          + [pltpu.SemaphoreType.DMA] * (2 * NBUF)
      ),
      compiler_params=pltpu.CompilerParams(use_tc_tiling_on_sc=False),
  )
  def msg_kernel(h_hbm, e_hbm, out_hbm,
                 flatv, srcv, dstv, rowsv, stagev, accum, *sems):
    gsem = sems[:NBUF]
    ssem = sems[NBUF:]
    c = lax.axis_index("c")
    s = lax.axis_index("s")
    wid = c * NS + s
    zero = jnp.zeros((lanes,), dtype)
    for r in range(KCH):
      for cc in range(w // lanes):
        stagev[r, pl.ds(cc * lanes, lanes)] = zero
    for off, sz in _chunks(rpt, KCH):
      pltpu.sync_copy(stagev.at[pl.ds(0, sz)],
                      accum.at[pl.ds(s * rpt + off, sz)])
    _load_repack(e_hbm, 0, epw, wid, flatv, srcv, wid * 16)
    _load_repack(e_hbm, 1, epw, wid, flatv, dstv, n)
    plsc.subcore_barrier()

    gd = [None] * NBUF
    sd = [None] * NBUF
    for j in range(min(PREF, ch)):
      gd[j % NBUF] = pltpu.async_copy(
          h_hbm.at[srcv.at[j]], rowsv.at[j % NBUF], gsem[j % NBUF])
    for j in range(ch):
      jn = j + PREF
      if jn < ch:
        bn = jn % NBUF
        if sd[bn] is not None:
          sd[bn].wait()
          sd[bn] = None
        gd[bn] = pltpu.async_copy(
            h_hbm.at[srcv.at[jn]], rowsv.at[bn], gsem[bn])
      b = j % NBUF
      gd[b].wait()
      sd[b] = pltpu.async_copy(
          rowsv.at[b], accum.at[dstv.at[j]], ssem[b], add=True)
    for b in range(NBUF):
      if sd[b] is not None:
        sd[b].wait()
    plsc.subcore_barrier()
    pltpu.sync_copy(accum.at[pl.ds(s * rpt, rpt)], stagev)
    pltpu.sync_copy(stagev, out_hbm.at[c, pl.ds(s * rpt, rpt)])

  return msg_kernel


def _mm1_body(x_ref, w_ref, degp_ref, h1s_ref, dinv_ref, *, n):
  deg = 1.0 + degp_ref[0, :n] + degp_ref[1, :n]
  dinv = lax.rsqrt(deg)[:, None]
  dinv_ref[...] = dinv
  h1 = jnp.dot(x_ref[...], w_ref[...], preferred_element_type=jnp.float32)
  h1s_ref[...] = (h1 * dinv).astype(h1s_ref.dtype)


def _mid_body(p_ref, h1_ref, dinv_ref, b1_ref, w2_ref, h2s_ref, *, n):
  out1 = (p_ref[0, :n, :].astype(jnp.float32)
          + p_ref[1, :n, :].astype(jnp.float32)
          + h1_ref[...].astype(jnp.float32))
  h2 = jnp.maximum(out1 * dinv_ref[...] + b1_ref[...], 0.0)
  h2s_ref[...] = jnp.dot(h2, w2_ref[...],
                         preferred_element_type=jnp.float32) * dinv_ref[...]


def _heads_body(mri_ref, cog_ref, clin_ref, gen_ref,
                wm_ref, bm_ref, wc_ref, bc_ref, wcl_ref, bcl_ref,
                wg_ref, bg_ref, mm_ref):
  mf = jnp.maximum(jnp.dot(mri_ref[...], wm_ref[...],
                           preferred_element_type=jnp.float32) + bm_ref[...], 0.0)
  cf = jnp.maximum(jnp.dot(cog_ref[...], wc_ref[...],
                           preferred_element_type=jnp.float32) + bc_ref[...], 0.0)
  clf = jnp.maximum(jnp.dot(clin_ref[...], wcl_ref[...],
                            preferred_element_type=jnp.float32) + bcl_ref[...], 0.0)
  gnf = jnp.maximum(jnp.dot(gen_ref[...], wg_ref[...],
                            preferred_element_type=jnp.float32) + bg_ref[...], 0.0)
  mm_ref[...] = jnp.concatenate([mf, cf, clf, gnf], axis=1)


def _final_body(q_ref, h2_ref, dinv_ref, b2_ref, batch_ref, mm_ref,
                wf1_ref, bf1_ref, wf2_ref, bf2_ref,
                o_ref, *, n, nb):
  out2 = q_ref[0, :n, :] + q_ref[1, :n, :] + h2_ref[...]
  gf = jnp.maximum(out2 * dinv_ref[...] + b2_ref[...], 0.0)
  seg = lax.broadcasted_iota(jnp.int32, (nb, n), 0)
  maskf = (seg == batch_ref[...]).astype(jnp.float32)
  sums = jnp.dot(maskf, gf, preferred_element_type=jnp.float32)
  cnt = jnp.sum(maskf, axis=1, keepdims=True)
  graph_emb = sums / jnp.maximum(cnt, 1.0)
  comb = jnp.concatenate([graph_emb, mm_ref[...]], axis=1)
  hid = jnp.maximum(jnp.dot(comb, wf1_ref[...],
                            preferred_element_type=jnp.float32) + bf1_ref[...], 0.0)
  logits = jnp.dot(hid, wf2_ref[...],
                   preferred_element_type=jnp.float32) + bf2_ref[...]
  m = jnp.max(logits, axis=1, keepdims=True)
  z = logits - m
  lse = jnp.log(jnp.sum(jnp.exp(z), axis=1, keepdims=True))
  o_ref[...] = z - lse


def kernel(x, edge_index, batch, mri_features, cog_features, clin_features,
           genetic_features, W1, b1, W2, b2, Wm, bm, Wc, bc, Wcl, bcl,
           Wg, bg, Wf1, bf1, Wf2, bf2):
  n, _ = x.shape
  e = edge_index.shape[1]
  nb = mri_features.shape[0]
  nr, _ = _pad_rows(n)

  if e % NW != 0 or (e // NW) % 8 != 0:
    epw = -(-e // NW)
    epw += (-epw) % 8
    pad = NW * epw - e
    pad_col = jnp.stack([jnp.arange(pad, dtype=jnp.int32) % n,
                         n + (jnp.arange(pad, dtype=jnp.int32) % 16)])
    edge_index = jnp.concatenate([edge_index, pad_col], axis=1)
    e = NW * epw

  d = x.shape[1]
  mm = pl.pallas_call(
      _heads_body,
      out_shape=jax.ShapeDtypeStruct((nb, 32), jnp.float32),
  )(mri_features, cog_features, clin_features, genetic_features,
    Wm, bm.reshape(1, -1), Wc, bc.reshape(1, -1), Wcl, bcl.reshape(1, -1),
    Wg, bg.reshape(1, -1))

  degp = _make_deg_kernel(n, e)(edge_index).reshape(2, nr)

  H1, dinv = pl.pallas_call(
      functools.partial(_mm1_body, n=n),
      out_shape=(jax.ShapeDtypeStruct((n, 32), jnp.bfloat16),
                 jax.ShapeDtypeStruct((n, 1), jnp.float32)),
  )(x, W1, degp)

  P = _make_msg_kernel(n, 32, e, jnp.bfloat16)(H1, edge_index)
  H2 = pl.pallas_call(
      functools.partial(_mid_body, n=n),
      out_shape=jax.ShapeDtypeStruct((n, 16), jnp.float32),
  )(P, H1, dinv, b1.reshape(1, 32), W2)

  Q = _make_msg_kernel(n, 16, e)(H2, edge_index)
  out = pl.pallas_call(
      functools.partial(_final_body, n=n, nb=nb),
      out_shape=jax.ShapeDtypeStruct((nb, 3), jnp.float32),
  )(Q, H2, dinv, b2.reshape(1, 16), batch.reshape(1, n), mm,
    Wf1, bf1.reshape(1, -1), Wf2, bf2.reshape(1, -1))
  return out
